# baseline (speedup 1.0000x reference)
import jax
import jax.numpy as jnp
from jax import lax
from jax.experimental import pallas as pl
from jax.experimental.pallas import tpu as pltpu
from jax.experimental.pallas import tpu_sc as plsc

_ITERATIONS = 16
_B = 256
_W = 8192
_LANES = 16
_WORDS = _W // 32
_HALO = 8
_PBUF = _HALO + _WORDS + _HALO
_NUM_WORKERS = 32
_PIECES = 2
_BP = _B // _PIECES
_RPW = _BP // _NUM_WORKERS
_WCHUNKS = _WORDS // _LANES


def _make_body(piece_base):
    return lambda *refs: _piece_body(piece_base, *refs)


def _piece_body(piece_base, in_hbm, lut_hbm, out_hbm, lut_v, in_v, pba, pbb,
                stg, sem_lo, sem_hi, sem_q, sem_in):
    wid = lax.axis_index("s") * 2 + lax.axis_index("c")

    pltpu.sync_copy(lut_hbm, lut_v)

    tblv = lut_v[...]
    masks = []
    for k in range(8):
        sk = jnp.where(tblv[k] >= 0.5, jnp.int32(-1), jnp.int32(0))
        masks.append(lax.broadcast_in_dim(sk, (_LANES,), ()))
    m0, m1, m2, m3, m4, m5, m6, m7 = masks

    zeros = jnp.zeros((_LANES,), jnp.int32)
    for buf in (pba, pbb):
        buf[pl.ds(0, _LANES)] = zeros
        buf[pl.ds(_PBUF - _LANES, _LANES)] = zeros

    lane_iota = lax.broadcasted_iota(jnp.int32, (_LANES,), 0)
    bit_lo = jnp.int32(1) << lane_iota
    bit_hi = bit_lo << 16
    perms = [lane_iota ^ s for s in (1, 2, 4, 8)]

    def lane_or(v):
        for pm in perms:
            v = v | v.at[pm].get(mode="promise_in_bounds")
        return v

    def fire_half(p, b, q, sem):
        pltpu.async_copy(
            stg.at[pl.ds(p, 4), :], out_hbm.at[b, pl.ds(q, 4), :], sem)

    def drain_half(b, q, sem):
        pltpu.make_async_copy(
            stg.at[pl.ds(0, 4), :], out_hbm.at[b, pl.ds(q, 4), :],
            sem).wait()

    def fire_last(b, sem):
        pltpu.async_copy(
            stg.at[pl.ds(0, 1), :], out_hbm.at[b, pl.ds(16, 1), :], sem)

    def drain_last(b, sem):
        pltpu.make_async_copy(
            stg.at[pl.ds(0, 1), :], out_hbm.at[b, pl.ds(16, 1), :],
            sem).wait()

    b0 = wid * _RPW
    fire_half(4, b0, 12, sem_hi)
    fire_last(b0, sem_q)

    def unpack(src, itm):
        @plsc.parallel_loop(0, _WORDS, unroll=4)
        def unp(w):
            wv = src[pl.ds(_HALO + (w & ~15), _LANES)]
            sel = lax.broadcast_in_dim(w & 15, (_LANES,), ())
            splat = wv.at[sel].get(mode="promise_in_bounds")
            lo = (lax.shift_right_logical(splat, lane_iota) & 1)
            hi = (lax.shift_right_logical(splat, lane_iota + 16) & 1)
            stg[itm, pl.ds(w * 32, _LANES)] = lo.astype(jnp.float32)
            stg[itm, pl.ds(w * 32 + 16, _LANES)] = hi.astype(jnp.float32)

    def row_body(rr, carry):
        b = wid * _RPW + rr
        pltpu.async_copy(
            in_hbm.at[pl.ds(piece_base + b, 1), :], in_v, sem_in).wait()

        drain_last(b, sem_q)

        @plsc.parallel_loop(0, _WCHUNKS)
        def packw(t):
            vec = jnp.zeros((_LANES,), jnp.int32)
            for u in range(_LANES):
                base = t * 512 + u * 32
                va = in_v[0, pl.ds(base, _LANES)]
                vb = in_v[0, pl.ds(base + _LANES, _LANES)]
                sa = jnp.where(va >= 0.5, 1.0, 0.0)
                sb = jnp.where(vb >= 0.5, 1.0, 0.0)
                stg[0, pl.ds(base, _LANES)] = sa
                stg[0, pl.ds(base + _LANES, _LANES)] = sb
                y = jnp.where(va >= 0.5, bit_lo, 0) | jnp.where(
                    vb >= 0.5, bit_hi, 0)
                y = lane_or(y)
                vec = jnp.where(lane_iota == u, y, vec)
            pba[pl.ds(_HALO + t * _LANES, _LANES)] = vec

        src, dst = pba, pbb
        for k in range(1, _ITERATIONS + 1):
            if k == 4:
                drain_half(b, 12, sem_hi)
            elif k == 8:
                drain_half(b, 0, sem_lo)
            elif k == 12:
                drain_half(b, 4, sem_hi)
            elif k == 16:
                drain_half(b, 8, sem_lo)

            @plsc.parallel_loop(0, _WCHUNKS, unroll=4)
            def chunk(t):
                base = t * _LANES
                p = src[pl.ds(_HALO - 1 + base, _LANES)]
                c = src[pl.ds(_HALO + base, _LANES)]
                n = src[pl.ds(_HALO + 1 + base, _LANES)]
                l = (c << 1) | lax.shift_right_logical(p, 31)
                r = lax.shift_right_logical(c, 1) | (n << 31)
                nl, nc, nr = ~l, ~c, ~r
                a0 = (l & m1) | (nl & m0)
                a1 = (l & m3) | (nl & m2)
                a2 = (l & m5) | (nl & m4)
                a3 = (l & m7) | (nl & m6)
                v0 = (c & a1) | (nc & a0)
                v1 = (c & a3) | (nc & a2)
                out = (r & v1) | (nr & v0)
                dst[pl.ds(_HALO + base, _LANES)] = out

            unpack(dst, k % 8)

            if k == 3:
                fire_half(0, b, 0, sem_lo)
            elif k == 7:
                fire_half(4, b, 4, sem_hi)
            elif k == 11:
                fire_half(0, b, 8, sem_lo)
            elif k == 15:
                fire_half(4, b, 12, sem_hi)
            elif k == 16:
                fire_last(b, sem_q)
            src, dst = dst, src
        return carry

    lax.fori_loop(0, _RPW, row_body, 0)

    blast = wid * _RPW + _RPW - 1
    drain_half(blast, 12, sem_hi)
    drain_last(blast, sem_q)


def _run_piece(piece, x, lut16):
    mesh = plsc.VectorSubcoreMesh(core_axis_name="c", subcore_axis_name="s")
    return pl.kernel(
        _make_body(piece * _BP),
        out_type=jax.ShapeDtypeStruct((_BP, _ITERATIONS + 1, _W), jnp.float32),
        mesh=mesh,
        scratch_types=[
            pltpu.VMEM((_LANES,), jnp.float32),
            pltpu.VMEM((1, _W), jnp.float32),
            pltpu.VMEM((_PBUF,), jnp.int32),
            pltpu.VMEM((_PBUF,), jnp.int32),
            pltpu.VMEM((8, _W), jnp.float32),
            pltpu.SemaphoreType.DMA,
            pltpu.SemaphoreType.DMA,
            pltpu.SemaphoreType.DMA,
            pltpu.SemaphoreType.DMA,
        ],
        name="ca_piece%d" % piece,
    )(x, lut16)


def kernel(input, lookup):
    lut16 = jnp.concatenate([lookup, jnp.zeros((8,), jnp.float32)])
    pieces = [_run_piece(p, input, lut16) for p in range(_PIECES)]
    return jnp.concatenate(pieces, axis=0)


# speedup vs baseline: 1.5219x; 1.5219x over previous
import jax
import jax.numpy as jnp
from jax import lax
from jax.experimental import pallas as pl
from jax.experimental.pallas import tpu as pltpu
from jax.experimental.pallas import tpu_sc as plsc

_ITERATIONS = 16
_B = 256
_W = 8192
_LANES = 16
_WORDS = _W // 32
_HALO = 8
_PBUF = _HALO + _WORDS + _HALO
_NUM_WORKERS = 32
_RPW = _B // _NUM_WORKERS
_WCHUNKS = _WORDS // _LANES


def _body(in_hbm, lut_hbm, out_hbm, lut_v, in_v, pba, pbb,
          stg, sem_lo, sem_hi, sem_q, sem_in):
    wid = lax.axis_index("s") * 2 + lax.axis_index("c")

    pltpu.sync_copy(lut_hbm, lut_v)

    tblv = lut_v[...]
    masks = []
    for k in range(8):
        sk = jnp.where(tblv[k] >= 0.5, jnp.int32(-1), jnp.int32(0))
        masks.append(lax.broadcast_in_dim(sk, (_LANES,), ()))
    m0, m1, m2, m3, m4, m5, m6, m7 = masks

    zeros = jnp.zeros((_LANES,), jnp.int32)
    for buf in (pba, pbb):
        buf[pl.ds(0, _LANES)] = zeros
        buf[pl.ds(_PBUF - _LANES, _LANES)] = zeros

    lane_iota = lax.broadcasted_iota(jnp.int32, (_LANES,), 0)
    bit_lo = jnp.int32(1) << lane_iota
    bit_hi = bit_lo << 16
    perms = [lane_iota ^ s for s in (1, 2, 4, 8)]

    def lane_or(v):
        for pm in perms:
            v = v | v.at[pm].get(mode="promise_in_bounds")
        return v

    def fire_half(p, b, q, sem):
        pltpu.async_copy(
            stg.at[pl.ds(p, 4), :], out_hbm.at[b, pl.ds(q, 4), :], sem)

    def drain_half(b, q, sem):
        pltpu.make_async_copy(
            stg.at[pl.ds(0, 4), :], out_hbm.at[b, pl.ds(q, 4), :],
            sem).wait()

    def fire_last(b, sem):
        pltpu.async_copy(
            stg.at[pl.ds(0, 1), :], out_hbm.at[b, pl.ds(16, 1), :], sem)

    def drain_last(b, sem):
        pltpu.make_async_copy(
            stg.at[pl.ds(0, 1), :], out_hbm.at[b, pl.ds(16, 1), :],
            sem).wait()

    b0 = wid * _RPW
    fire_half(4, b0, 12, sem_hi)
    fire_last(b0, sem_q)

    def unpack(src, itm):
        @plsc.parallel_loop(0, _WCHUNKS)
        def unp(t):
            wv = src[pl.ds(_HALO + t * _LANES, _LANES)]
            for u in range(_LANES):
                sel = jnp.full((_LANES,), u, jnp.int32)
                splat = wv.at[sel].get(mode="promise_in_bounds")
                lo = (lax.shift_right_logical(splat, lane_iota) & 1)
                hi = (lax.shift_right_logical(splat, lane_iota + 16) & 1)
                base = (t * _LANES + u) * 32
                stg[itm, pl.ds(base, _LANES)] = lo.astype(jnp.float32)
                stg[itm, pl.ds(base + 16, _LANES)] = hi.astype(jnp.float32)

    def row_body(rr, carry):
        b = wid * _RPW + rr
        pltpu.async_copy(in_hbm.at[pl.ds(b, 1), :], in_v, sem_in).wait()

        drain_last(b, sem_q)

        @plsc.parallel_loop(0, _WCHUNKS)
        def packw(t):
            vec = jnp.zeros((_LANES,), jnp.int32)
            for u in range(_LANES):
                base = t * 512 + u * 32
                va = in_v[0, pl.ds(base, _LANES)]
                vb = in_v[0, pl.ds(base + _LANES, _LANES)]
                sa = jnp.where(va >= 0.5, 1.0, 0.0)
                sb = jnp.where(vb >= 0.5, 1.0, 0.0)
                stg[0, pl.ds(base, _LANES)] = sa
                stg[0, pl.ds(base + _LANES, _LANES)] = sb
                y = jnp.where(va >= 0.5, bit_lo, 0) | jnp.where(
                    vb >= 0.5, bit_hi, 0)
                y = lane_or(y)
                vec = jnp.where(lane_iota == u, y, vec)
            pba[pl.ds(_HALO + t * _LANES, _LANES)] = vec

        src, dst = pba, pbb
        for k in range(1, _ITERATIONS + 1):
            if k == 4:
                drain_half(b, 12, sem_hi)
            elif k == 8:
                drain_half(b, 0, sem_lo)
            elif k == 12:
                drain_half(b, 4, sem_hi)
            elif k == 16:
                drain_half(b, 8, sem_lo)

            @plsc.parallel_loop(0, _WCHUNKS, unroll=4)
            def chunk(t):
                base = t * _LANES
                p = src[pl.ds(_HALO - 1 + base, _LANES)]
                c = src[pl.ds(_HALO + base, _LANES)]
                n = src[pl.ds(_HALO + 1 + base, _LANES)]
                l = (c << 1) | lax.shift_right_logical(p, 31)
                r = lax.shift_right_logical(c, 1) | (n << 31)
                nl, nc, nr = ~l, ~c, ~r
                a0 = (l & m1) | (nl & m0)
                a1 = (l & m3) | (nl & m2)
                a2 = (l & m5) | (nl & m4)
                a3 = (l & m7) | (nl & m6)
                v0 = (c & a1) | (nc & a0)
                v1 = (c & a3) | (nc & a2)
                out = (r & v1) | (nr & v0)
                dst[pl.ds(_HALO + base, _LANES)] = out

            unpack(dst, k % 8)

            if k == 3:
                fire_half(0, b, 0, sem_lo)
            elif k == 7:
                fire_half(4, b, 4, sem_hi)
            elif k == 11:
                fire_half(0, b, 8, sem_lo)
            elif k == 15:
                fire_half(4, b, 12, sem_hi)
            elif k == 16:
                fire_last(b, sem_q)
            src, dst = dst, src
        return carry

    lax.fori_loop(0, _RPW, row_body, 0)

    blast = wid * _RPW + _RPW - 1
    drain_half(blast, 12, sem_hi)
    drain_last(blast, sem_q)


def _run(x, lut16):
    mesh = plsc.VectorSubcoreMesh(core_axis_name="c", subcore_axis_name="s")
    return pl.kernel(
        _body,
        out_type=jax.ShapeDtypeStruct((_B, _ITERATIONS + 1, _W), jnp.float32),
        mesh=mesh,
        scratch_types=[
            pltpu.VMEM((_LANES,), jnp.float32),
            pltpu.VMEM((1, _W), jnp.float32),
            pltpu.VMEM((_PBUF,), jnp.int32),
            pltpu.VMEM((_PBUF,), jnp.int32),
            pltpu.VMEM((8, _W), jnp.float32),
            pltpu.SemaphoreType.DMA,
            pltpu.SemaphoreType.DMA,
            pltpu.SemaphoreType.DMA,
            pltpu.SemaphoreType.DMA,
        ],
    )(x, lut16)


def kernel(input, lookup):
    lut16 = jnp.concatenate([lookup, jnp.zeros((8,), jnp.float32)])
    return _run(input, lut16)
